# baseline (speedup 1.0000x reference)
import functools

import jax
import jax.numpy as jnp
from jax.experimental import pallas as pl
from jax.experimental.pallas import tpu as pltpu

K = 50
BR = 128
TC = 128
NEG_INF = float("-inf")


def _topk_body(xr_ref, xc_ref, vals_ref, idxs_ref, s_ref):
    i = pl.program_id(0)
    nt = i + 1
    r0 = i * BR
    ntiles_max = s_ref.shape[1] // TC

    row_in_tile = jax.lax.broadcasted_iota(jnp.int32, (BR, TC), 0)
    col_in_tile = jax.lax.broadcasted_iota(jnp.int32, (BR, TC), 1)
    tile_iota = jax.lax.broadcasted_iota(jnp.int32, (BR, ntiles_max), 1)

    def build(t, tmaxes):
        a = xr_ref[:, pl.ds(t * TC, TC)]
        b = xc_ref[pl.ds(t * TC, TC), :]
        rows = r0 + row_in_tile
        cols = t * TC + col_in_tile
        s = jnp.where(cols < rows, a + b.T, NEG_INF)
        s_ref[:, pl.ds(t * TC, TC)] = s
        tm = jnp.max(s, axis=1, keepdims=True)
        return jnp.where(tile_iota == t, tm, tmaxes)

    tmaxes0 = jnp.full((BR, ntiles_max), NEG_INF, jnp.float32)
    tmaxes = jax.lax.fori_loop(0, nt, build, tmaxes0)

    big = 1 << 30

    def extract(k, carry):
        vals, idxs, tmaxes = carry
        m = jnp.max(tmaxes, axis=1, keepdims=True)

        def locate(t, acc):
            tile = s_ref[:, pl.ds(t * TC, TC)]
            cols = t * TC + col_in_tile
            cand = jnp.min(jnp.where(tile == m, cols, big), axis=1,
                           keepdims=True)
            return jnp.minimum(acc, cand)

        j = jax.lax.fori_loop(0, nt, locate, jnp.full((BR, 1), big,
                                                      jnp.int32))

        def mask(t, tmaxes):
            tile = s_ref[:, pl.ds(t * TC, TC)]
            cols = t * TC + col_in_tile
            tile = jnp.where(cols == j, NEG_INF, tile)
            s_ref[:, pl.ds(t * TC, TC)] = tile
            tm = jnp.max(tile, axis=1, keepdims=True)
            return jnp.where(tile_iota == t, tm, tmaxes)

        tmaxes = jax.lax.fori_loop(0, nt, mask, tmaxes)

        kk = jax.lax.broadcasted_iota(jnp.int32, (BR, 64), 1) == k
        vals = jnp.where(kk, m, vals)
        idxs = jnp.where(kk, j, idxs)
        return vals, idxs, tmaxes

    vals0 = jnp.full((BR, 64), NEG_INF, jnp.float32)
    idxs0 = jnp.zeros((BR, 64), jnp.int32)
    vals, idxs, _ = jax.lax.fori_loop(0, K, extract,
                                      (vals0, idxs0, tmaxes))

    rows = r0 + jax.lax.broadcasted_iota(jnp.int32, (BR, K), 0)
    p = jax.lax.broadcasted_iota(jnp.int32, (BR, K), 1)
    pad = p >= rows
    vals_ref[...] = jnp.where(pad, NEG_INF, vals[:, :K])
    idxs_ref[...] = jnp.where(pad, p, idxs[:, :K])


@functools.partial(jax.jit, static_argnames=())
def kernel(mentions):
    n = mentions.shape[0]
    nblocks = n // BR
    ntiles = n // TC
    grid = (nblocks,)
    out_vals = jax.ShapeDtypeStruct((n, K), jnp.float32)
    out_idxs = jax.ShapeDtypeStruct((n, K), jnp.int32)
    return pl.pallas_call(
        _topk_body,
        grid=grid,
        in_specs=[
            pl.BlockSpec((BR, n), lambda i: (i, 0)),
            pl.BlockSpec((n, BR), lambda i: (0, i)),
        ],
        out_specs=[
            pl.BlockSpec((BR, K), lambda i: (i, 0)),
            pl.BlockSpec((BR, K), lambda i: (i, 0)),
        ],
        out_shape=[out_vals, out_idxs],
        scratch_shapes=[
            pltpu.VMEM((BR, n), jnp.float32),
        ],
    )(mentions, mentions)


# speedup vs baseline: 5.1598x; 5.1598x over previous
import functools

import jax
import jax.numpy as jnp
from jax import lax
from jax.experimental import pallas as pl
from jax.experimental.pallas import tpu as pltpu
from jax.experimental.pallas import tpu_sc as plsc

K = 50
N = 8192
NW = 32
RPW = N // NW
PADK = 128
NEG = float("-inf")
BIG = 1 << 30
RB = 256
CAPA = 544
SEG = 256
BOOT = 64
ROWPAD = N + SEG

BT = 512


def _ssum_body(a_ref, b_ref, o_ref):
    o_ref[...] = a_ref[...] + b_ref[...].T


def _ssum(m):
    g = N // BT
    return pl.pallas_call(
        _ssum_body,
        grid=(g, g),
        in_specs=[
            pl.BlockSpec((BT, BT), lambda i, j: (i, j)),
            pl.BlockSpec((BT, BT), lambda i, j: (j, i)),
        ],
        out_specs=pl.BlockSpec((BT, BT), lambda i, j: (i, j)),
        out_shape=jax.ShapeDtypeStruct((N, N), jnp.float32),
    )(m, m)


def _iota16():
    return lax.iota(jnp.int32, 16)


def _topk_sc(ssum):
    mesh = plsc.VectorSubcoreMesh(core_axis_name="c", subcore_axis_name="s")

    @functools.partial(
        pl.kernel,
        mesh=mesh,
        out_type=[
            jax.ShapeDtypeStruct((N, PADK), jnp.float32),
            jax.ShapeDtypeStruct((N, PADK), jnp.int32),
        ],
        scratch_types=[
            pltpu.VMEM((ROWPAD,), jnp.float32),
            pltpu.VMEM((ROWPAD,), jnp.float32),
            pltpu.VMEM((CAPA,), jnp.float32),
            pltpu.VMEM((CAPA,), jnp.int32),
            pltpu.VMEM((PADK,), jnp.float32),
            pltpu.VMEM((PADK,), jnp.int32),
            pltpu.SemaphoreType.DMA,
            pltpu.SemaphoreType.DMA,
        ],
        compiler_params=pltpu.CompilerParams(needs_layout_passes=False),
    )
    def k(s_hbm, ov_hbm, oi_hbm, row0, row1, bufv, bufi, outv, outi,
          sem0, sem1):
        wid = lax.axis_index("s") * 2 + lax.axis_index("c")
        lane0 = _iota16() == 0

        def extract(ptr):
            bufv[pl.ds(ptr, 16)] = jnp.full((16,), NEG, jnp.float32)
            ngrp = ptr // 16 + 1

            def one(kk, _):
                def scan(g, c):
                    bv, bs, bc = c
                    x = bufv[pl.ds(g * 16, 16)]
                    ci = bufi[pl.ds(g * 16, 16)]
                    sl = g * 16 + _iota16()
                    m = x > bv
                    return (jnp.where(m, x, bv), jnp.where(m, sl, bs),
                            jnp.where(m, ci, bc))

                bv, bs, bc = lax.fori_loop(
                    0, ngrp, scan,
                    (jnp.full((16,), NEG, jnp.float32),
                     jnp.full((16,), BIG, jnp.int32),
                     jnp.zeros((16,), jnp.int32)))
                mx = jnp.max(bv)
                tie = bv == mx
                smin = jnp.minimum(jnp.min(jnp.where(tie, bs, BIG)),
                                   jnp.int32(CAPA - 17))
                col = jnp.min(jnp.where(bs == smin, bc, BIG))
                plsc.store_scatter(bufv, [jnp.full((16,), smin, jnp.int32)],
                                   jnp.full((16,), NEG, jnp.float32), mask=lane0)
                kk16 = jnp.full((16,), kk, jnp.int32)
                plsc.store_scatter(outv, [kk16],
                                   jnp.full((16,), mx, jnp.float32), mask=lane0)
                plsc.store_scatter(outi, [kk16],
                                   jnp.full((16,), col, jnp.int32), mask=lane0)
                return mx

            return lax.fori_loop(0, K, one, jnp.float32(NEG))

        def rebuild(ptr):
            thr = extract(ptr)
            for g in range(4):
                bufv[pl.ds(g * 16, 16)] = outv[pl.ds(g * 16, 16)]
                bufi[pl.ds(g * 16, 16)] = outi[pl.ds(g * 16, 16)]
            return thr, jnp.int32(K)

        def append(x, cols, mask, ptr):
            cnt = mask.astype(jnp.int32)
            pos = ptr + plsc.cumsum(cnt) - 1
            plsc.store_scatter(bufv, [pos], x, mask=mask)
            plsc.store_scatter(bufi, [pos], cols, mask=mask)
            return ptr + jnp.sum(cnt)

        def process(i, row):
            thr = jnp.float32(NEG)
            ptr = jnp.int32(0)
            for g in range(BOOT // 16):
                x = row[pl.ds(g * 16, 16)]
                cols = g * 16 + _iota16()
                mask = jnp.logical_and(cols < i, x > thr)
                ptr = append(x, cols, mask, ptr)
            thr, ptr = rebuild(ptr)

            nseg = jnp.maximum(0, (i - BOOT + SEG - 1) // SEG)

            def seg_body(s, c):
                thr, ptr = c
                base = BOOT + s * SEG
                for g in range(SEG // 16):
                    x = row[pl.ds(base + g * 16, 16)]
                    cols = base + g * 16 + _iota16()
                    mask = jnp.logical_and(cols < i, x > thr)
                    ptr = append(x, cols, mask, ptr)
                return lax.cond(ptr >= RB, lambda: rebuild(ptr),
                                lambda: (thr, ptr))

            thr, ptr = lax.fori_loop(0, nseg, seg_body, (thr, ptr))
            extract(ptr)

            for g in range(PADK // 16):
                pv = g * 16 + _iota16()
                pm = pv >= i
                outv[pl.ds(g * 16, 16)] = jnp.where(
                    pm, jnp.full((16,), NEG, jnp.float32),
                    outv[pl.ds(g * 16, 16)])
                outi[pl.ds(g * 16, 16)] = jnp.where(
                    pm, pv, outi[pl.ds(g * 16, 16)])
            pltpu.sync_copy(outv, ov_hbm.at[i])
            pltpu.sync_copy(outi, oi_hbm.at[i])

        pltpu.async_copy(s_hbm.at[wid], row0.at[pl.ds(0, N)], sem0)

        def pair(rp, carry):
            i0 = wid + NW * 2 * rp
            i1 = i0 + NW
            pltpu.async_copy(s_hbm.at[i1], row1.at[pl.ds(0, N)], sem1)
            pltpu.make_async_copy(s_hbm.at[i0], row0.at[pl.ds(0, N)],
                                  sem0).wait()
            process(i0, row0)

            @pl.when(2 * rp + 2 < RPW)
            def _():
                pltpu.async_copy(s_hbm.at[i1 + NW], row0.at[pl.ds(0, N)],
                                 sem0)

            pltpu.make_async_copy(s_hbm.at[i1], row1.at[pl.ds(0, N)],
                                  sem1).wait()
            process(i1, row1)
            return carry

        lax.fori_loop(0, RPW // 2, pair, 0)

    return k(ssum)


@jax.jit
def kernel(mentions):
    s = _ssum(mentions)
    vals, idxs = _topk_sc(s)
    return vals[:, :K], idxs[:, :K]
